# speedup vs baseline: 3.7974x; 3.7974x over previous
import functools

import jax
import jax.numpy as jnp
from jax import lax
from jax.experimental import pallas as pl
from jax.experimental.pallas import tpu as pltpu
from jax.experimental.pallas import tpu_sc as plsc

F32 = jnp.float32


def _tables_body(nf_ref, wes_ref, wed_ref, u_ref, weu_ref, be_ref,
                 hs_ref, hd_ref):
    nf = nf_ref[...]
    hs_ref[...] = jnp.dot(nf, wes_ref[...], preferred_element_type=F32)
    cvec = jnp.dot(u_ref[...], weu_ref[...], preferred_element_type=F32) + be_ref[...]
    hd_ref[...] = jnp.dot(nf, wed_ref[...], preferred_element_type=F32) + cvec


def _ebase_body(ef_ref, wee_ref, out_ref):
    out_ref[...] = jnp.dot(ef_ref[...], wee_ref[...], preferred_element_type=F32)


def _make_sc_edge_kernel(E, N, DH, NC, NS, K):
    NW = NC * NS
    nchunk = E // K
    trips = (nchunk + NW - 1) // NW
    mesh = plsc.VectorSubcoreMesh(core_axis_name="c", subcore_axis_name="s",
                                  num_cores=NC, num_subcores=NS)

    @functools.partial(
        pl.kernel,
        out_type=[jax.ShapeDtypeStruct((E, DH), F32),
                  jax.ShapeDtypeStruct((NC, N, DH), F32)],
        mesh=mesh,
        scratch_types=[
            pltpu.VMEM((K,), jnp.int32),
            pltpu.VMEM((K,), jnp.int32),
            pltpu.VMEM((K, DH), F32),
            pltpu.VMEM((K, DH), F32),
            pltpu.VMEM((K, DH), F32),
            pltpu.VMEM_SHARED((N, DH), F32),
            pltpu.SemaphoreType.DMA,
        ],
    )
    def sc_edge(eb_hbm, hs_hbm, hd_hbm, src_hbm, dst_hbm, zero_hbm,
                upd_hbm, agg_hbm,
                src_v, dst_v, rs_v, rd_v, eb_v, agg_sp, sem):
        c = lax.axis_index("c")
        s = lax.axis_index("s")
        wid = s * NC + c

        @pl.when(s == 0)
        def _():
            pltpu.sync_copy(zero_hbm, agg_sp)

        plsc.subcore_barrier()

        def trip(i, carry):
            cid = wid + NW * i

            @pl.when(cid < nchunk)
            def _():
                base = cid * K
                pltpu.sync_copy(src_hbm.at[pl.ds(base, K)], src_v)
                pltpu.sync_copy(dst_hbm.at[pl.ds(base, K)], dst_v)
                cp_s = pltpu.async_copy(hs_hbm.at[src_v], rs_v, sem)
                cp_d = pltpu.async_copy(hd_hbm.at[dst_v], rd_v, sem)
                cp_e = pltpu.async_copy(eb_hbm.at[pl.ds(base, K)], eb_v, sem)
                cp_s.wait()
                cp_d.wait()
                cp_e.wait()

                def row(r, rcarry):
                    for j in range(DH // 16):
                        sl = pl.ds(j * 16, 16)
                        rs_v[r, sl] = rs_v[r, sl] + rd_v[r, sl] + eb_v[r, sl]
                    return rcarry

                lax.fori_loop(0, K, row, 0)
                pltpu.sync_copy(rs_v, upd_hbm.at[pl.ds(base, K)])
                pltpu.sync_copy(rs_v, agg_sp.at[dst_v], add=True)

            return carry

        lax.fori_loop(0, trips, trip, 0)
        plsc.subcore_barrier()

        @pl.when(s == 0)
        def _():
            pltpu.sync_copy(agg_sp, agg_hbm.at[c])

    return sc_edge


def _final_body(p_ref, nf_ref, u_ref, wna_ref, wnn_ref, wnu_ref, bn_ref,
                wge_ref, wgn_ref, wgu_ref, bg_ref, unf_ref, uu_ref):
    agg = p_ref[0] + p_ref[1]
    u = u_ref[...]
    unf = (jnp.dot(agg, wna_ref[...], preferred_element_type=F32)
           + jnp.dot(nf_ref[...], wnn_ref[...], preferred_element_type=F32)
           + jnp.dot(u, wnu_ref[...], preferred_element_type=F32)
           + bn_ref[...])
    unf_ref[...] = unf
    edge_aggr = jnp.sum(agg, axis=0, keepdims=True)
    node_aggr = jnp.sum(unf, axis=0, keepdims=True)
    uu_ref[...] = (jnp.dot(edge_aggr, wge_ref[...], preferred_element_type=F32)
                   + jnp.dot(node_aggr, wgn_ref[...], preferred_element_type=F32)
                   + jnp.dot(u, wgu_ref[...], preferred_element_type=F32)
                   + bg_ref[...])


def kernel(nf, ef, u, edge_index, We, be, Wn, bn, Wg, bg):
    N, DN = nf.shape
    E, DE = ef.shape
    DG = u.shape[1]
    DH = We.shape[1]
    NC, NS = 2, 16
    K = 128

    src = edge_index[0]
    dst = edge_index[1]
    We_e = We[:DE]
    We_s = We[DE:DE + DN]
    We_d = We[DE + DN:DE + 2 * DN]
    We_u = We[DE + 2 * DN:]
    Wn_a = Wn[:DH]
    Wn_n = Wn[DH:DH + DN]
    Wn_u = Wn[DH + DN:]
    Wg_e = Wg[:DH]
    Wg_n = Wg[DH:DH + DN]
    Wg_u = Wg[DH + DN:]
    be2 = be.reshape(1, DH)
    bn2 = bn.reshape(1, DN)
    bg2 = bg.reshape(1, DG)

    h_s, h_d2 = pl.pallas_call(
        _tables_body,
        out_shape=[jax.ShapeDtypeStruct((N, DH), F32),
                   jax.ShapeDtypeStruct((N, DH), F32)],
    )(nf, We_s, We_d, u, We_u, be2)

    BE = 8000
    e_base = pl.pallas_call(
        _ebase_body,
        grid=(E // BE,),
        in_specs=[pl.BlockSpec((BE, DE), lambda i: (i, 0)),
                  pl.BlockSpec((DE, DH), lambda i: (0, 0))],
        out_specs=pl.BlockSpec((BE, DH), lambda i: (i, 0)),
        out_shape=jax.ShapeDtypeStruct((E, DH), F32),
    )(ef, We_e)

    zero = jnp.zeros((N, DH), dtype=F32)
    sc_edge = _make_sc_edge_kernel(E, N, DH, NC, NS, K)
    updated_ef, agg_parts = sc_edge(e_base, h_s, h_d2, src, dst, zero)

    updated_nf, updated_u = pl.pallas_call(
        _final_body,
        out_shape=[jax.ShapeDtypeStruct((N, DN), F32),
                   jax.ShapeDtypeStruct((1, DG), F32)],
    )(agg_parts, nf, u, Wn_a, Wn_n, Wn_u, bn2, Wg_e, Wg_n, Wg_u, bg2)

    return updated_nf, updated_ef, updated_u


# speedup vs baseline: 3.9671x; 1.0447x over previous
import functools

import jax
import jax.numpy as jnp
from jax import lax
from jax.experimental import pallas as pl
from jax.experimental.pallas import tpu as pltpu
from jax.experimental.pallas import tpu_sc as plsc

F32 = jnp.float32


def _tables_body(nf_ref, wes_ref, wed_ref, u_ref, weu_ref, be_ref,
                 hs_ref, hd_ref):
    nf = nf_ref[...]
    hs_ref[...] = jnp.dot(nf, wes_ref[...], preferred_element_type=F32)
    cvec = jnp.dot(u_ref[...], weu_ref[...], preferred_element_type=F32) + be_ref[...]
    hd_ref[...] = jnp.dot(nf, wed_ref[...], preferred_element_type=F32) + cvec


def _ebase_body(ef_ref, wee_ref, out_ref):
    out_ref[...] = jnp.dot(ef_ref[...], wee_ref[...], preferred_element_type=F32)


def _make_sc_edge_kernel(E, N, DH, NC, NS, K):
    NW = NC * NS
    nchunk = E // K
    trips = (nchunk + NW - 1) // NW
    mesh = plsc.VectorSubcoreMesh(core_axis_name="c", subcore_axis_name="s",
                                  num_cores=NC, num_subcores=NS)

    @functools.partial(
        pl.kernel,
        out_type=[jax.ShapeDtypeStruct((E, DH), F32),
                  jax.ShapeDtypeStruct((NC, N, DH), F32)],
        mesh=mesh,
        scratch_types=[
            pltpu.VMEM((K,), jnp.int32),
            pltpu.VMEM((K,), jnp.int32),
            pltpu.VMEM((K, DH), F32),
            pltpu.VMEM((K, DH), F32),
            pltpu.VMEM((K, DH), F32),
            pltpu.VMEM_SHARED((N, DH), F32),
            pltpu.SemaphoreType.DMA,
            pltpu.SemaphoreType.DMA,
            pltpu.SemaphoreType.DMA,
        ],
    )
    def sc_edge(eb_hbm, hs_hbm, hd_hbm, src_hbm, dst_hbm, zero_hbm,
                upd_hbm, agg_hbm,
                src_v, dst_v, rs_v, rd_v, eb_v, agg_sp, sem_in, sem_st,
                sem_sc):
        c = lax.axis_index("c")
        s = lax.axis_index("s")
        wid = s * NC + c

        @pl.when(s == 0)
        def _():
            pltpu.sync_copy(zero_hbm, agg_sp)

        plsc.subcore_barrier()

        def trip(i, carry):
            cid = wid + NW * i

            @pl.when(cid < nchunk)
            def _():
                base = cid * K
                pltpu.sync_copy(src_hbm.at[pl.ds(base, K)], src_v)
                pltpu.sync_copy(dst_hbm.at[pl.ds(base, K)], dst_v)
                cp_s = pltpu.async_copy(hs_hbm.at[src_v], rs_v, sem_in)
                cp_d = pltpu.async_copy(hd_hbm.at[dst_v], rd_v, sem_in)
                cp_e = pltpu.async_copy(eb_hbm.at[pl.ds(base, K)], eb_v, sem_in)
                cp_s.wait()
                cp_d.wait()
                cp_e.wait()

                def row(r, rcarry):
                    for j in range(DH // 16):
                        sl = pl.ds(j * 16, 16)
                        rs_v[r, sl] = rs_v[r, sl] + rd_v[r, sl] + eb_v[r, sl]
                    return rcarry

                lax.fori_loop(0, K, row, 0)
                o1 = pltpu.async_copy(rs_v, upd_hbm.at[pl.ds(base, K)], sem_st)
                o2 = pltpu.async_copy(rs_v, agg_sp.at[dst_v], sem_sc, add=True)
                o1.wait()
                o2.wait()

            return carry

        lax.fori_loop(0, trips, trip, 0)
        plsc.subcore_barrier()

        @pl.when(s == 0)
        def _():
            pltpu.sync_copy(agg_sp, agg_hbm.at[c])

    return sc_edge


def _final_body(p_ref, nf_ref, u_ref, wna_ref, wnn_ref, wnu_ref, bn_ref,
                wge_ref, wgn_ref, wgu_ref, bg_ref, unf_ref, uu_ref):
    agg = p_ref[0] + p_ref[1]
    u = u_ref[...]
    unf = (jnp.dot(agg, wna_ref[...], preferred_element_type=F32)
           + jnp.dot(nf_ref[...], wnn_ref[...], preferred_element_type=F32)
           + jnp.dot(u, wnu_ref[...], preferred_element_type=F32)
           + bn_ref[...])
    unf_ref[...] = unf
    edge_aggr = jnp.sum(agg, axis=0, keepdims=True)
    node_aggr = jnp.sum(unf, axis=0, keepdims=True)
    uu_ref[...] = (jnp.dot(edge_aggr, wge_ref[...], preferred_element_type=F32)
                   + jnp.dot(node_aggr, wgn_ref[...], preferred_element_type=F32)
                   + jnp.dot(u, wgu_ref[...], preferred_element_type=F32)
                   + bg_ref[...])


def kernel(nf, ef, u, edge_index, We, be, Wn, bn, Wg, bg):
    N, DN = nf.shape
    E, DE = ef.shape
    DG = u.shape[1]
    DH = We.shape[1]
    NC, NS = 2, 16
    K = 128

    src = edge_index[0]
    dst = edge_index[1]
    We_e = We[:DE]
    We_s = We[DE:DE + DN]
    We_d = We[DE + DN:DE + 2 * DN]
    We_u = We[DE + 2 * DN:]
    Wn_a = Wn[:DH]
    Wn_n = Wn[DH:DH + DN]
    Wn_u = Wn[DH + DN:]
    Wg_e = Wg[:DH]
    Wg_n = Wg[DH:DH + DN]
    Wg_u = Wg[DH + DN:]
    be2 = be.reshape(1, DH)
    bn2 = bn.reshape(1, DN)
    bg2 = bg.reshape(1, DG)

    h_s, h_d2 = pl.pallas_call(
        _tables_body,
        out_shape=[jax.ShapeDtypeStruct((N, DH), F32),
                   jax.ShapeDtypeStruct((N, DH), F32)],
    )(nf, We_s, We_d, u, We_u, be2)

    BE = 8000
    e_base = pl.pallas_call(
        _ebase_body,
        grid=(E // BE,),
        in_specs=[pl.BlockSpec((BE, DE), lambda i: (i, 0)),
                  pl.BlockSpec((DE, DH), lambda i: (0, 0))],
        out_specs=pl.BlockSpec((BE, DH), lambda i: (i, 0)),
        out_shape=jax.ShapeDtypeStruct((E, DH), F32),
    )(ef, We_e)

    zero = jnp.zeros((N, DH), dtype=F32)
    sc_edge = _make_sc_edge_kernel(E, N, DH, NC, NS, K)
    updated_ef, agg_parts = sc_edge(e_base, h_s, h_d2, src, dst, zero)

    updated_nf, updated_u = pl.pallas_call(
        _final_body,
        out_shape=[jax.ShapeDtypeStruct((N, DN), F32),
                   jax.ShapeDtypeStruct((1, DG), F32)],
    )(agg_parts, nf, u, Wn_a, Wn_n, Wn_u, bn2, Wg_e, Wg_n, Wg_u, bg2)

    return updated_nf, updated_ef, updated_u


# speedup vs baseline: 5.3161x; 1.3401x over previous
import functools

import jax
import jax.numpy as jnp
from jax import lax
from jax.experimental import pallas as pl
from jax.experimental.pallas import tpu as pltpu
from jax.experimental.pallas import tpu_sc as plsc

F32 = jnp.float32


def _tables_body(nf_ref, wes_ref, wed_ref, u_ref, weu_ref, be_ref,
                 hs_ref, hd_ref):
    nf = nf_ref[...]
    hs_ref[...] = jnp.dot(nf, wes_ref[...], preferred_element_type=F32)
    cvec = jnp.dot(u_ref[...], weu_ref[...], preferred_element_type=F32) + be_ref[...]
    hd_ref[...] = jnp.dot(nf, wed_ref[...], preferred_element_type=F32) + cvec


def _ebase_body(ef_ref, wee_ref, out_ref):
    out_ref[...] = jnp.dot(ef_ref[...], wee_ref[...], preferred_element_type=F32)


def _make_sc_edge_kernel(E, N, DH, NC, NS, K):
    NW = NC * NS
    trips = E // (NW * K)
    assert E == NW * trips * K and trips % 4 == 2 and trips >= 6
    mesh = plsc.VectorSubcoreMesh(core_axis_name="c", subcore_axis_name="s",
                                  num_cores=NC, num_subcores=NS)

    @functools.partial(
        pl.kernel,
        out_type=[jax.ShapeDtypeStruct((E, DH), F32),
                  jax.ShapeDtypeStruct((NC, N, DH), F32)],
        mesh=mesh,
        scratch_types=[
            [pltpu.VMEM((K,), jnp.int32) for _ in range(4)],
            [pltpu.VMEM((K,), jnp.int32) for _ in range(4)],
            [pltpu.VMEM((K, DH), F32) for _ in range(2)],
            [pltpu.VMEM((K, DH), F32) for _ in range(2)],
            [pltpu.VMEM((K, DH), F32) for _ in range(2)],
            pltpu.VMEM_SHARED((N, DH), F32),
            [pltpu.SemaphoreType.DMA for _ in range(4)],
            [pltpu.SemaphoreType.DMA for _ in range(2)],
            [pltpu.SemaphoreType.DMA for _ in range(2)],
            [pltpu.SemaphoreType.DMA for _ in range(2)],
        ],
    )
    def sc_edge(eb_hbm, hs_hbm, hd_hbm, src_hbm, dst_hbm, zero_hbm,
                upd_hbm, agg_hbm,
                sidx, didx, rs, rd, eb, agg_sp,
                sem_idx, sem_in, sem_st, sem_sc):
        c = lax.axis_index("c")
        s = lax.axis_index("s")
        wid = s * NC + c

        def fire_idx(cid, q):
            pltpu.async_copy(src_hbm.at[wid, cid], sidx[q], sem_idx[q])
            pltpu.async_copy(dst_hbm.at[wid, cid], didx[q], sem_idx[q])

        def wait_idx(cid, q):
            pltpu.make_async_copy(src_hbm.at[wid, cid], sidx[q], sem_idx[q]).wait()
            pltpu.make_async_copy(dst_hbm.at[wid, cid], didx[q], sem_idx[q]).wait()

        def fire_rows(cid, b, q):
            base = (wid * trips + cid) * K
            pltpu.async_copy(hs_hbm.at[sidx[q]], rs[b], sem_in[b])
            pltpu.async_copy(hd_hbm.at[didx[q]], rd[b], sem_in[b])
            pltpu.async_copy(eb_hbm.at[pl.ds(base, K)], eb[b], sem_in[b])

        def wait_rows(cid, b, q):
            base = (wid * trips + cid) * K
            pltpu.make_async_copy(hs_hbm.at[sidx[q]], rs[b], sem_in[b]).wait()
            pltpu.make_async_copy(hd_hbm.at[didx[q]], rd[b], sem_in[b]).wait()
            pltpu.make_async_copy(eb_hbm.at[pl.ds(base, K)], eb[b],
                                  sem_in[b]).wait()

        def fire_out(cid, b, q):
            base = (wid * trips + cid) * K
            pltpu.async_copy(rs[b], upd_hbm.at[pl.ds(base, K)], sem_st[b])
            pltpu.async_copy(rs[b], agg_sp.at[didx[q]], sem_sc[b], add=True)

        def drain_out(cid, b, q):
            base = (wid * trips + cid) * K
            pltpu.make_async_copy(rs[b], upd_hbm.at[pl.ds(base, K)],
                                  sem_st[b]).wait()
            pltpu.make_async_copy(rs[b], agg_sp.at[didx[q]], sem_sc[b]).wait()

        def compute(b):
            rsb, rdb, ebb = rs[b], rd[b], eb[b]

            def row(r, rcarry):
                for j in range(DH // 16):
                    sl = pl.ds(j * 16, 16)
                    rsb[r, sl] = rsb[r, sl] + rdb[r, sl] + ebb[r, sl]
                return rcarry

            lax.fori_loop(0, K, row, 0)

        @pl.when(s == 0)
        def _():
            pltpu.sync_copy(zero_hbm, agg_sp)

        fire_idx(0, 0)
        wait_idx(0, 0)
        fire_idx(1, 1)
        fire_rows(0, 0, 0)
        plsc.subcore_barrier()

        def step(cid, j, t):
            b = j % 2
            q = j % 4

            def drain():
                drain_out(cid - 1, 1 - b, (j + 3) % 4)

            if j == 0:
                @pl.when(t >= 1)
                def _():
                    drain()
            else:
                drain()
            fire_idx(cid + 2, (j + 2) % 4)
            wait_idx(cid + 1, (j + 1) % 4)
            fire_rows(cid + 1, 1 - b, (j + 1) % 4)
            wait_rows(cid, b, q)
            compute(b)
            fire_out(cid, b, q)

        def quad(t, carry):
            for j in range(4):
                step(4 * t + j, j, t)
            return carry

        nq = (trips - 2) // 4
        lax.fori_loop(0, nq, quad, 0)
        c0 = trips - 2
        drain_out(c0 - 1, 1, 3)
        wait_idx(c0 + 1, 1)
        fire_rows(c0 + 1, 1, 1)
        wait_rows(c0, 0, 0)
        compute(0)
        fire_out(c0, 0, 0)
        drain_out(c0, 0, 0)
        wait_rows(c0 + 1, 1, 1)
        compute(1)
        fire_out(c0 + 1, 1, 1)
        drain_out(c0 + 1, 1, 1)

        plsc.subcore_barrier()

        @pl.when(s == 0)
        def _():
            pltpu.sync_copy(agg_sp, agg_hbm.at[c])

    return sc_edge


def _final_body(p_ref, nf_ref, u_ref, wna_ref, wnn_ref, wnu_ref, bn_ref,
                wge_ref, wgn_ref, wgu_ref, bg_ref, unf_ref, uu_ref):
    agg = p_ref[0] + p_ref[1]
    u = u_ref[...]
    unf = (jnp.dot(agg, wna_ref[...], preferred_element_type=F32)
           + jnp.dot(nf_ref[...], wnn_ref[...], preferred_element_type=F32)
           + jnp.dot(u, wnu_ref[...], preferred_element_type=F32)
           + bn_ref[...])
    unf_ref[...] = unf
    edge_aggr = jnp.sum(agg, axis=0, keepdims=True)
    node_aggr = jnp.sum(unf, axis=0, keepdims=True)
    uu_ref[...] = (jnp.dot(edge_aggr, wge_ref[...], preferred_element_type=F32)
                   + jnp.dot(node_aggr, wgn_ref[...], preferred_element_type=F32)
                   + jnp.dot(u, wgu_ref[...], preferred_element_type=F32)
                   + bg_ref[...])


def kernel(nf, ef, u, edge_index, We, be, Wn, bn, Wg, bg):
    N, DN = nf.shape
    E, DE = ef.shape
    DG = u.shape[1]
    DH = We.shape[1]
    NC, NS = 2, 16
    K = 40
    NW = NC * NS
    trips = E // (NW * K)

    src = edge_index[0].reshape(NW, trips, K)
    dst = edge_index[1].reshape(NW, trips, K)
    We_e = We[:DE]
    We_s = We[DE:DE + DN]
    We_d = We[DE + DN:DE + 2 * DN]
    We_u = We[DE + 2 * DN:]
    Wn_a = Wn[:DH]
    Wn_n = Wn[DH:DH + DN]
    Wn_u = Wn[DH + DN:]
    Wg_e = Wg[:DH]
    Wg_n = Wg[DH:DH + DN]
    Wg_u = Wg[DH + DN:]
    be2 = be.reshape(1, DH)
    bn2 = bn.reshape(1, DN)
    bg2 = bg.reshape(1, DG)

    h_s, h_d2 = pl.pallas_call(
        _tables_body,
        out_shape=[jax.ShapeDtypeStruct((N, DH), F32),
                   jax.ShapeDtypeStruct((N, DH), F32)],
    )(nf, We_s, We_d, u, We_u, be2)

    BE = 8000
    e_base = pl.pallas_call(
        _ebase_body,
        grid=(E // BE,),
        in_specs=[pl.BlockSpec((BE, DE), lambda i: (i, 0)),
                  pl.BlockSpec((DE, DH), lambda i: (0, 0))],
        out_specs=pl.BlockSpec((BE, DH), lambda i: (i, 0)),
        out_shape=jax.ShapeDtypeStruct((E, DH), F32),
    )(ef, We_e)

    zero = jnp.zeros((N, DH), dtype=F32)
    sc_edge = _make_sc_edge_kernel(E, N, DH, NC, NS, K)
    updated_ef, agg_parts = sc_edge(e_base, h_s, h_d2, src, dst, zero)

    updated_nf, updated_u = pl.pallas_call(
        _final_body,
        out_shape=[jax.ShapeDtypeStruct((N, DN), F32),
                   jax.ShapeDtypeStruct((1, DG), F32)],
    )(agg_parts, nf, u, Wn_a, Wn_n, Wn_u, bn2, Wg_e, Wg_n, Wg_u, bg2)

    return updated_nf, updated_ef, updated_u


# speedup vs baseline: 5.3768x; 1.0114x over previous
import functools

import jax
import jax.numpy as jnp
from jax import lax
from jax.experimental import pallas as pl
from jax.experimental.pallas import tpu as pltpu
from jax.experimental.pallas import tpu_sc as plsc

F32 = jnp.float32


def _tables_body(nf_ref, wes_ref, wed_ref, u_ref, weu_ref, be_ref,
                 hs_ref, hd_ref):
    nf = nf_ref[...]
    hs_ref[...] = jnp.dot(nf, wes_ref[...], preferred_element_type=F32)
    cvec = jnp.dot(u_ref[...], weu_ref[...], preferred_element_type=F32) + be_ref[...]
    hd_ref[...] = jnp.dot(nf, wed_ref[...], preferred_element_type=F32) + cvec


def _ebase_body(ef_ref, wee_ref, out_ref):
    y = jnp.dot(ef_ref[...], wee_ref[...], preferred_element_type=F32)
    half = y.shape[1] // 2
    lo = lax.bitcast_convert_type(y[:, :half].astype(jnp.bfloat16),
                                  jnp.uint16).astype(jnp.uint32)
    hi = lax.bitcast_convert_type(y[:, half:].astype(jnp.bfloat16),
                                  jnp.uint16).astype(jnp.uint32)
    out_ref[...] = lax.bitcast_convert_type(lo | (hi << 16), jnp.int32)


def _make_sc_edge_kernel(E, N, DH, NC, NS, K):
    NW = NC * NS
    trips = E // (NW * K)
    assert E == NW * trips * K and trips % 4 == 2 and trips >= 6
    mesh = plsc.VectorSubcoreMesh(core_axis_name="c", subcore_axis_name="s",
                                  num_cores=NC, num_subcores=NS)

    @functools.partial(
        pl.kernel,
        out_type=[jax.ShapeDtypeStruct((E, DH), F32),
                  jax.ShapeDtypeStruct((NC, N, DH), F32)],
        mesh=mesh,
        scratch_types=[
            [pltpu.VMEM((K,), jnp.int32) for _ in range(4)],
            [pltpu.VMEM((K,), jnp.int32) for _ in range(4)],
            [pltpu.VMEM((K, DH), F32) for _ in range(2)],
            [pltpu.VMEM((K, DH), F32) for _ in range(2)],
            [pltpu.VMEM((K, DH // 2), jnp.int32) for _ in range(2)],
            pltpu.VMEM_SHARED((N, DH), F32),
            [pltpu.SemaphoreType.DMA for _ in range(4)],
            [pltpu.SemaphoreType.DMA for _ in range(2)],
            [pltpu.SemaphoreType.DMA for _ in range(2)],
            [pltpu.SemaphoreType.DMA for _ in range(2)],
        ],
    )
    def sc_edge(eb_hbm, hs_hbm, hd_hbm, src_hbm, dst_hbm, zero_hbm,
                upd_hbm, agg_hbm,
                sidx, didx, rs, rd, eb, agg_sp,
                sem_idx, sem_in, sem_st, sem_sc):
        c = lax.axis_index("c")
        s = lax.axis_index("s")
        wid = s * NC + c

        def fire_idx(cid, q):
            pltpu.async_copy(src_hbm.at[wid, cid], sidx[q], sem_idx[q])
            pltpu.async_copy(dst_hbm.at[wid, cid], didx[q], sem_idx[q])

        def wait_idx(cid, q):
            pltpu.make_async_copy(src_hbm.at[wid, cid], sidx[q], sem_idx[q]).wait()
            pltpu.make_async_copy(dst_hbm.at[wid, cid], didx[q], sem_idx[q]).wait()

        def fire_rows(cid, b, q):
            base = (wid * trips + cid) * K
            pltpu.async_copy(hs_hbm.at[sidx[q]], rs[b], sem_in[b])
            pltpu.async_copy(hd_hbm.at[didx[q]], rd[b], sem_in[b])
            pltpu.async_copy(eb_hbm.at[pl.ds(base, K)], eb[b], sem_in[b])

        def wait_rows(cid, b, q):
            base = (wid * trips + cid) * K
            pltpu.make_async_copy(hs_hbm.at[sidx[q]], rs[b], sem_in[b]).wait()
            pltpu.make_async_copy(hd_hbm.at[didx[q]], rd[b], sem_in[b]).wait()
            pltpu.make_async_copy(eb_hbm.at[pl.ds(base, K)], eb[b],
                                  sem_in[b]).wait()

        def fire_out(cid, b, q):
            base = (wid * trips + cid) * K
            pltpu.async_copy(rs[b], upd_hbm.at[pl.ds(base, K)], sem_st[b])
            pltpu.async_copy(rs[b], agg_sp.at[didx[q]], sem_sc[b], add=True)

        def drain_out(cid, b, q):
            base = (wid * trips + cid) * K
            pltpu.make_async_copy(rs[b], upd_hbm.at[pl.ds(base, K)],
                                  sem_st[b]).wait()
            pltpu.make_async_copy(rs[b], agg_sp.at[didx[q]], sem_sc[b]).wait()

        def compute(b):
            rsb, rdb, ebb = rs[b], rd[b], eb[b]

            himask = jnp.full((16,), -65536, dtype=jnp.int32)

            def row(r, rcarry):
                for j in range(DH // 32):
                    w = ebb[r, pl.ds(j * 16, 16)]
                    e0 = lax.bitcast_convert_type(w << 16, F32)
                    e1 = lax.bitcast_convert_type(w & himask, F32)
                    sl0 = pl.ds(j * 32, 16)
                    sl1 = pl.ds(j * 32 + 16, 16)
                    rsb[r, sl0] = rsb[r, sl0] + rdb[r, sl0] + e0
                    rsb[r, sl1] = rsb[r, sl1] + rdb[r, sl1] + e1
                return rcarry

            lax.fori_loop(0, K, row, 0)

        @pl.when(s == 0)
        def _():
            pltpu.sync_copy(zero_hbm, agg_sp)

        fire_idx(0, 0)
        wait_idx(0, 0)
        fire_idx(1, 1)
        fire_rows(0, 0, 0)
        plsc.subcore_barrier()

        def step(cid, j, t):
            b = j % 2
            q = j % 4

            def drain():
                drain_out(cid - 1, 1 - b, (j + 3) % 4)

            if j == 0:
                @pl.when(t >= 1)
                def _():
                    drain()
            else:
                drain()
            fire_idx(cid + 2, (j + 2) % 4)
            wait_idx(cid + 1, (j + 1) % 4)
            fire_rows(cid + 1, 1 - b, (j + 1) % 4)
            wait_rows(cid, b, q)
            compute(b)
            fire_out(cid, b, q)

        def quad(t, carry):
            for j in range(4):
                step(4 * t + j, j, t)
            return carry

        nq = (trips - 2) // 4
        lax.fori_loop(0, nq, quad, 0)
        c0 = trips - 2
        drain_out(c0 - 1, 1, 3)
        wait_idx(c0 + 1, 1)
        fire_rows(c0 + 1, 1, 1)
        wait_rows(c0, 0, 0)
        compute(0)
        fire_out(c0, 0, 0)
        drain_out(c0, 0, 0)
        wait_rows(c0 + 1, 1, 1)
        compute(1)
        fire_out(c0 + 1, 1, 1)
        drain_out(c0 + 1, 1, 1)

        plsc.subcore_barrier()

        @pl.when(s == 0)
        def _():
            pltpu.sync_copy(agg_sp, agg_hbm.at[c])

    return sc_edge


def _final_body(p_ref, nf_ref, u_ref, wna_ref, wnn_ref, wnu_ref, bn_ref,
                wge_ref, wgn_ref, wgu_ref, bg_ref, unf_ref, uu_ref):
    agg = p_ref[0] + p_ref[1]
    u = u_ref[...]
    unf = (jnp.dot(agg, wna_ref[...], preferred_element_type=F32)
           + jnp.dot(nf_ref[...], wnn_ref[...], preferred_element_type=F32)
           + jnp.dot(u, wnu_ref[...], preferred_element_type=F32)
           + bn_ref[...])
    unf_ref[...] = unf
    edge_aggr = jnp.sum(agg, axis=0, keepdims=True)
    node_aggr = jnp.sum(unf, axis=0, keepdims=True)
    uu_ref[...] = (jnp.dot(edge_aggr, wge_ref[...], preferred_element_type=F32)
                   + jnp.dot(node_aggr, wgn_ref[...], preferred_element_type=F32)
                   + jnp.dot(u, wgu_ref[...], preferred_element_type=F32)
                   + bg_ref[...])


def kernel(nf, ef, u, edge_index, We, be, Wn, bn, Wg, bg):
    N, DN = nf.shape
    E, DE = ef.shape
    DG = u.shape[1]
    DH = We.shape[1]
    NC, NS = 2, 16
    K = 40
    NW = NC * NS
    trips = E // (NW * K)

    src = edge_index[0].reshape(NW, trips, K)
    dst = edge_index[1].reshape(NW, trips, K)
    We_e = We[:DE]
    We_s = We[DE:DE + DN]
    We_d = We[DE + DN:DE + 2 * DN]
    We_u = We[DE + 2 * DN:]
    Wn_a = Wn[:DH]
    Wn_n = Wn[DH:DH + DN]
    Wn_u = Wn[DH + DN:]
    Wg_e = Wg[:DH]
    Wg_n = Wg[DH:DH + DN]
    Wg_u = Wg[DH + DN:]
    be2 = be.reshape(1, DH)
    bn2 = bn.reshape(1, DN)
    bg2 = bg.reshape(1, DG)

    h_s, h_d2 = pl.pallas_call(
        _tables_body,
        out_shape=[jax.ShapeDtypeStruct((N, DH), F32),
                   jax.ShapeDtypeStruct((N, DH), F32)],
    )(nf, We_s, We_d, u, We_u, be2)

    half = DH // 2
    perm = ([32 * (m // 16) + m % 16 for m in range(half)]
            + [32 * (m // 16) + 16 + m % 16 for m in range(half)])
    We_e_il = We_e[:, jnp.array(perm, dtype=jnp.int32)]

    BE = 8000
    e_base = pl.pallas_call(
        _ebase_body,
        grid=(E // BE,),
        in_specs=[pl.BlockSpec((BE, DE), lambda i: (i, 0)),
                  pl.BlockSpec((DE, DH), lambda i: (0, 0))],
        out_specs=pl.BlockSpec((BE, half), lambda i: (i, 0)),
        out_shape=jax.ShapeDtypeStruct((E, half), jnp.int32),
    )(ef, We_e_il)

    zero = jnp.zeros((N, DH), dtype=F32)
    sc_edge = _make_sc_edge_kernel(E, N, DH, NC, NS, K)
    updated_ef, agg_parts = sc_edge(e_base, h_s, h_d2, src, dst, zero)

    updated_nf, updated_u = pl.pallas_call(
        _final_body,
        out_shape=[jax.ShapeDtypeStruct((N, DN), F32),
                   jax.ShapeDtypeStruct((1, DG), F32)],
    )(agg_parts, nf, u, Wn_a, Wn_n, Wn_u, bn2, Wg_e, Wg_n, Wg_u, bg2)

    return updated_nf, updated_ef, updated_u
